# baseline (speedup 1.0000x reference)
import functools

import jax
import jax.numpy as jnp
from jax import lax
from jax.experimental import pallas as pl
from jax.experimental.pallas import tpu as pltpu
from jax.experimental.pallas import tpu_sc as plsc

N = 10000
E = 320000
IN_DIM = 128
HID = 64
OUT_DIM = 128

NC = 2
NS = 16
NW = NC * NS
EPW = E // NW
CH = 128
NITER = 79
EPAD = NW * NITER * CH
PADROW = 10200
NPAD = 10240
RPT = NPAD // NS
DEGW = 8

BLK = 1000
GRID = N // BLK

_SC_MESH = plsc.VectorSubcoreMesh(core_axis_name="c", subcore_axis_name="s")


@functools.partial(
    pl.kernel,
    out_type=(
        jax.ShapeDtypeStruct((NC * NPAD, HID), jnp.float32),
        jax.ShapeDtypeStruct((NC * NPAD, DEGW), jnp.float32),
    ),
    mesh=_SC_MESH,
    compiler_params=pltpu.CompilerParams(use_tc_tiling_on_sc=False),
    scratch_types=[
        pltpu.VMEM((NITER, CH), jnp.int32),
        pltpu.VMEM((NITER, CH), jnp.int32),
        pltpu.VMEM((2, CH, HID), jnp.float32),
        pltpu.VMEM((CH, DEGW), jnp.float32),
        pltpu.VMEM_SHARED((NPAD, HID), jnp.float32),
        pltpu.VMEM_SHARED((NPAD, DEGW), jnp.float32),
        pltpu.SemaphoreType.DMA,
        pltpu.SemaphoreType.DMA,
    ],
)
def _sc_agg_deg(xp, edge4, z64, z16, ones16, part_out, degp_out,
                sidx, didx, rows, ones_v, acc, dacc, sem0, sem1):
    c = lax.axis_index("c")
    s = lax.axis_index("s")
    wid = c * NS + s
    r0 = pl.multiple_of(s * RPT, 8)

    pltpu.sync_copy(edge4.at[0].at[wid], sidx)
    pltpu.sync_copy(edge4.at[1].at[wid], didx)
    pltpu.async_copy(xp.at[sidx.at[0]], rows.at[0], sem0)
    pltpu.sync_copy(z64.at[pl.ds(r0, RPT)], acc.at[pl.ds(r0, RPT)])
    pltpu.sync_copy(z16.at[pl.ds(r0, RPT)], dacc.at[pl.ds(r0, RPT)])
    pltpu.sync_copy(ones16, ones_v)
    plsc.subcore_barrier()

    def scat(i, buf):
        pltpu.sync_copy(rows.at[buf], acc.at[didx.at[i]], add=True)
        pltpu.sync_copy(ones_v, dacc.at[didx.at[i]], add=True)

    def step(k, carry):
        i0 = 2 * k
        g1 = pltpu.async_copy(xp.at[sidx.at[i0 + 1]], rows.at[1], sem1)
        pltpu.make_async_copy(xp.at[sidx.at[i0]], rows.at[0], sem0).wait()
        scat(i0, 0)
        pltpu.async_copy(xp.at[sidx.at[i0 + 2]], rows.at[0], sem0)
        g1.wait()
        scat(i0 + 1, 1)
        return carry

    lax.fori_loop(0, (NITER - 1) // 2, step, 0)
    pltpu.make_async_copy(xp.at[sidx.at[NITER - 1]], rows.at[0], sem0).wait()
    scat(NITER - 1, 0)
    plsc.subcore_barrier()

    out_r0 = pl.multiple_of(c * NPAD + s * RPT, 8)
    pltpu.sync_copy(acc.at[pl.ds(r0, RPT)], part_out.at[pl.ds(out_r0, RPT)])
    pltpu.sync_copy(dacc.at[pl.ds(r0, RPT)], degp_out.at[pl.ds(out_r0, RPT)])


@functools.partial(
    pl.kernel,
    out_type=jax.ShapeDtypeStruct((NC * NPAD, HID), jnp.float32),
    mesh=_SC_MESH,
    compiler_params=pltpu.CompilerParams(use_tc_tiling_on_sc=False),
    scratch_types=[
        pltpu.VMEM((NITER, CH), jnp.int32),
        pltpu.VMEM((NITER, CH), jnp.int32),
        pltpu.VMEM((2, CH, HID), jnp.float32),
        pltpu.VMEM_SHARED((NPAD, HID), jnp.float32),
        pltpu.SemaphoreType.DMA,
        pltpu.SemaphoreType.DMA,
    ],
)
def _sc_agg(xp, edge4, z64, part_out, sidx, didx, rows, acc, sem0, sem1):
    c = lax.axis_index("c")
    s = lax.axis_index("s")
    wid = c * NS + s
    r0 = pl.multiple_of(s * RPT, 8)

    pltpu.sync_copy(edge4.at[0].at[wid], sidx)
    pltpu.sync_copy(edge4.at[1].at[wid], didx)
    pltpu.async_copy(xp.at[sidx.at[0]], rows.at[0], sem0)
    pltpu.sync_copy(z64.at[pl.ds(r0, RPT)], acc.at[pl.ds(r0, RPT)])
    plsc.subcore_barrier()

    def scat(i, buf):
        pltpu.sync_copy(rows.at[buf], acc.at[didx.at[i]], add=True)

    def step(k, carry):
        i0 = 2 * k
        g1 = pltpu.async_copy(xp.at[sidx.at[i0 + 1]], rows.at[1], sem1)
        pltpu.make_async_copy(xp.at[sidx.at[i0]], rows.at[0], sem0).wait()
        scat(i0, 0)
        pltpu.async_copy(xp.at[sidx.at[i0 + 2]], rows.at[0], sem0)
        g1.wait()
        scat(i0 + 1, 1)
        return carry

    lax.fori_loop(0, (NITER - 1) // 2, step, 0)
    pltpu.make_async_copy(xp.at[sidx.at[NITER - 1]], rows.at[0], sem0).wait()
    scat(NITER - 1, 0)
    plsc.subcore_barrier()

    out_r0 = pl.multiple_of(c * NPAD + s * RPT, 8)
    pltpu.sync_copy(acc.at[pl.ds(r0, RPT)], part_out.at[pl.ds(out_r0, RPT)])


def _tc_a1_body(x_ref, wl, xp_o):
    xp_o[...] = jnp.dot(x_ref[...], wl[...], preferred_element_type=jnp.float32)


def _tc_a2_body(x_ref, wr, wres, bl, br, bres, xr_o, xres_o):
    x = x_ref[...]
    xr_o[...] = jnp.dot(x, wr[...], preferred_element_type=jnp.float32) + bl[...] + br[...]
    xres_o[...] = jnp.dot(x, wres[...], preferred_element_type=jnp.float32) + bres[...]


def _ln_relu(h, g, be):
    mu = jnp.mean(h, axis=1, keepdims=True)
    var = jnp.mean((h - mu) * (h - mu), axis=1, keepdims=True)
    h = (h - mu) * lax.rsqrt(var + 1e-5) * g + be
    return jnp.maximum(h, 0.0)


def _tc_b1_body(pa, pb, da, db, xr, xres, g, be, wl2, x1_o, xp2_o):
    ssum = pa[0] + pb[0]
    deg = jnp.sum(da[0] + db[0], axis=1, keepdims=True)
    rdeg = 1.0 / jnp.maximum(deg, 1.0)
    h = ssum * rdeg + xr[...]
    h = _ln_relu(h, g[...], be[...])
    x1 = xres[...] + h
    x1_o[...] = x1
    xp2_o[...] = jnp.dot(x1, wl2[...], preferred_element_type=jnp.float32)


def _tc_b2_body(x1_ref, wr2, bl2, br2, xr2_o):
    xr2_o[...] = jnp.dot(x1_ref[...], wr2[...], preferred_element_type=jnp.float32) + bl2[...] + br2[...]


def _tc_c_body(pa, pb, da, db, xr2, x1, g, be, wh, bh, out_o):
    ssum = pa[0] + pb[0]
    deg = jnp.sum(da[0] + db[0], axis=1, keepdims=True)
    rdeg = 1.0 / jnp.maximum(deg, 1.0)
    h = ssum * rdeg + xr2[...]
    h = _ln_relu(h, g[...], be[...])
    x2 = x1[...] + h
    out_o[...] = jnp.dot(x2, wh[...], preferred_element_type=jnp.float32) + bh[...]


def _row_spec(width):
    return pl.BlockSpec((BLK, width), lambda i: (i, 0))


def _slab_spec(width, slab):
    return pl.BlockSpec((1, BLK, width), lambda i, _s=slab: (_s, i, 0))


def _full_spec(shape):
    return pl.BlockSpec(shape, lambda i: (0,) * len(shape))


def kernel(x, edge_index, Wl1, bl1, Wr1, br1, g1, be1, Wres1, bres1,
           Wl2, bl2, Wr2, br2, g2, be2, Wh, bh):
    epad = jnp.concatenate(
        [edge_index.astype(jnp.int32),
         jnp.broadcast_to(jnp.array([[0], [PADROW]], jnp.int32), (2, EPAD - E))],
        axis=1)
    edge4 = epad.reshape(2, NW, NITER, CH)

    z64 = jnp.zeros((NPAD, HID), jnp.float32)
    z16 = jnp.zeros((NPAD, DEGW), jnp.float32)
    ones16 = jnp.zeros((CH, DEGW), jnp.float32).at[:, 0].set(1.0)

    bl1r = bl1.reshape(1, HID)
    br1r = br1.reshape(1, HID)
    bres1r = bres1.reshape(1, HID)
    g1r = g1.reshape(1, HID)
    be1r = be1.reshape(1, HID)
    bl2r = bl2.reshape(1, HID)
    br2r = br2.reshape(1, HID)
    g2r = g2.reshape(1, HID)
    be2r = be2.reshape(1, HID)
    bhr = bh.reshape(1, OUT_DIM)

    xp1 = pl.pallas_call(
        _tc_a1_body,
        grid=(GRID,),
        in_specs=[_row_spec(IN_DIM), _full_spec((IN_DIM, HID))],
        out_specs=_row_spec(HID),
        out_shape=jax.ShapeDtypeStruct((N, HID), jnp.float32),
    )(x, Wl1)

    xr1, xres = pl.pallas_call(
        _tc_a2_body,
        grid=(GRID,),
        in_specs=[
            _row_spec(IN_DIM),
            _full_spec((IN_DIM, HID)),
            _full_spec((IN_DIM, HID)),
            _full_spec((1, HID)),
            _full_spec((1, HID)),
            _full_spec((1, HID)),
        ],
        out_specs=[_row_spec(HID), _row_spec(HID)],
        out_shape=[
            jax.ShapeDtypeStruct((N, HID), jnp.float32),
            jax.ShapeDtypeStruct((N, HID), jnp.float32),
        ],
    )(x, Wr1, Wres1, bl1r, br1r, bres1r)

    part1, degp = _sc_agg_deg(xp1, edge4, z64, z16, ones16)
    part1 = part1.reshape(NC, NPAD, HID)
    degp = degp.reshape(NC, NPAD, DEGW)

    x1, xp2 = pl.pallas_call(
        _tc_b1_body,
        grid=(GRID,),
        in_specs=[
            _slab_spec(HID, 0), _slab_spec(HID, 1),
            _slab_spec(DEGW, 0), _slab_spec(DEGW, 1),
            _row_spec(HID), _row_spec(HID),
            _full_spec((1, HID)), _full_spec((1, HID)),
            _full_spec((HID, HID)),
        ],
        out_specs=[_row_spec(HID), _row_spec(HID)],
        out_shape=[
            jax.ShapeDtypeStruct((N, HID), jnp.float32),
            jax.ShapeDtypeStruct((N, HID), jnp.float32),
        ],
    )(part1, part1, degp, degp, xr1, xres, g1r, be1r, Wl2)

    part2 = _sc_agg(xp2, edge4, z64)
    part2 = part2.reshape(NC, NPAD, HID)

    xr2 = pl.pallas_call(
        _tc_b2_body,
        grid=(GRID,),
        in_specs=[
            _row_spec(HID),
            _full_spec((HID, HID)),
            _full_spec((1, HID)),
            _full_spec((1, HID)),
        ],
        out_specs=_row_spec(HID),
        out_shape=jax.ShapeDtypeStruct((N, HID), jnp.float32),
    )(x1, Wr2, bl2r, br2r)

    out = pl.pallas_call(
        _tc_c_body,
        grid=(GRID,),
        in_specs=[
            _slab_spec(HID, 0), _slab_spec(HID, 1),
            _slab_spec(DEGW, 0), _slab_spec(DEGW, 1),
            _row_spec(HID), _row_spec(HID),
            _full_spec((1, HID)), _full_spec((1, HID)),
            _full_spec((HID, OUT_DIM)), _full_spec((1, OUT_DIM)),
        ],
        out_specs=_row_spec(OUT_DIM),
        out_shape=jax.ShapeDtypeStruct((N, OUT_DIM), jnp.float32),
    )(part2, part2, degp, degp, xr2, x1, g2r, be2r, Wh, bhr)

    return out


# speedup vs baseline: 1.7106x; 1.7106x over previous
import functools

import jax
import jax.numpy as jnp
from jax import lax
from jax.experimental import pallas as pl
from jax.experimental.pallas import tpu as pltpu
from jax.experimental.pallas import tpu_sc as plsc

N = 10000
E = 320000
IN_DIM = 128
HID = 64
OUT_DIM = 128

NC = 2
NS = 16
NW = NC * NS
EPW = E // NW
CH = 80
NITER = EPW // CH
NPAD = 10240
RPT = NPAD // NS
DEGW = 8

BLK = 1000
GRID = N // BLK

_SC_MESH = plsc.VectorSubcoreMesh(core_axis_name="c", subcore_axis_name="s")


@functools.partial(
    pl.kernel,
    out_type=(
        jax.ShapeDtypeStruct((NC * NPAD, HID), jnp.float32),
        jax.ShapeDtypeStruct((NC * NPAD, DEGW), jnp.float32),
    ),
    mesh=_SC_MESH,
    compiler_params=pltpu.CompilerParams(use_tc_tiling_on_sc=False),
    scratch_types=[
        pltpu.VMEM((NITER, CH), jnp.int32),
        pltpu.VMEM((NITER, CH), jnp.int32),
        pltpu.VMEM((3, CH, HID), jnp.float32),
        pltpu.VMEM((CH, DEGW), jnp.float32),
        pltpu.VMEM_SHARED((NPAD, HID), jnp.float32),
        pltpu.VMEM_SHARED((NPAD, DEGW), jnp.float32),
        pltpu.SemaphoreType.DMA,
        pltpu.SemaphoreType.DMA,
        pltpu.SemaphoreType.DMA,
    ],
)
def _sc_agg_deg(xp, edge4, z64, z16, ones16, part_out, degp_out,
                sidx, didx, rows, ones_v, acc, dacc, sem0, sem1, sem2):
    c = lax.axis_index("c")
    s = lax.axis_index("s")
    wid = c * NS + s
    r0 = pl.multiple_of(s * RPT, 8)

    pltpu.sync_copy(edge4.at[0].at[wid], sidx)
    pltpu.sync_copy(edge4.at[1].at[wid], didx)
    pltpu.async_copy(xp.at[sidx.at[0]], rows.at[0], sem0)
    pltpu.async_copy(xp.at[sidx.at[1]], rows.at[1], sem1)
    pltpu.sync_copy(z64.at[pl.ds(r0, RPT)], acc.at[pl.ds(r0, RPT)])
    pltpu.sync_copy(z16.at[pl.ds(r0, RPT)], dacc.at[pl.ds(r0, RPT)])
    pltpu.sync_copy(ones16, ones_v)
    plsc.subcore_barrier()

    def scat(i, buf):
        pltpu.sync_copy(rows.at[buf], acc.at[didx.at[i]], add=True)
        pltpu.sync_copy(ones_v, dacc.at[didx.at[i]], add=True)

    def step(k, carry):
        i = 3 * k
        pltpu.async_copy(xp.at[sidx.at[i + 2]], rows.at[2], sem2)
        pltpu.make_async_copy(xp.at[sidx.at[i]], rows.at[0], sem0).wait()
        scat(i, 0)
        pltpu.async_copy(xp.at[sidx.at[i + 3]], rows.at[0], sem0)
        pltpu.make_async_copy(xp.at[sidx.at[i + 1]], rows.at[1], sem1).wait()
        scat(i + 1, 1)
        pltpu.async_copy(xp.at[sidx.at[i + 4]], rows.at[1], sem1)
        pltpu.make_async_copy(xp.at[sidx.at[i + 2]], rows.at[2], sem2).wait()
        scat(i + 2, 2)
        return carry

    lax.fori_loop(0, (NITER - 2) // 3, step, 0)
    pltpu.make_async_copy(xp.at[sidx.at[NITER - 2]], rows.at[0], sem0).wait()
    scat(NITER - 2, 0)
    pltpu.make_async_copy(xp.at[sidx.at[NITER - 1]], rows.at[1], sem1).wait()
    scat(NITER - 1, 1)
    plsc.subcore_barrier()

    out_r0 = pl.multiple_of(c * NPAD + s * RPT, 8)
    pltpu.sync_copy(acc.at[pl.ds(r0, RPT)], part_out.at[pl.ds(out_r0, RPT)])
    pltpu.sync_copy(dacc.at[pl.ds(r0, RPT)], degp_out.at[pl.ds(out_r0, RPT)])


@functools.partial(
    pl.kernel,
    out_type=jax.ShapeDtypeStruct((NC * NPAD, HID), jnp.float32),
    mesh=_SC_MESH,
    compiler_params=pltpu.CompilerParams(use_tc_tiling_on_sc=False),
    scratch_types=[
        pltpu.VMEM((NITER, CH), jnp.int32),
        pltpu.VMEM((NITER, CH), jnp.int32),
        pltpu.VMEM((3, CH, HID), jnp.float32),
        pltpu.VMEM_SHARED((NPAD, HID), jnp.float32),
        pltpu.SemaphoreType.DMA,
        pltpu.SemaphoreType.DMA,
        pltpu.SemaphoreType.DMA,
    ],
)
def _sc_agg(xp, edge4, z64, part_out, sidx, didx, rows, acc, sem0, sem1, sem2):
    c = lax.axis_index("c")
    s = lax.axis_index("s")
    wid = c * NS + s
    r0 = pl.multiple_of(s * RPT, 8)

    pltpu.sync_copy(edge4.at[0].at[wid], sidx)
    pltpu.sync_copy(edge4.at[1].at[wid], didx)
    pltpu.async_copy(xp.at[sidx.at[0]], rows.at[0], sem0)
    pltpu.async_copy(xp.at[sidx.at[1]], rows.at[1], sem1)
    pltpu.sync_copy(z64.at[pl.ds(r0, RPT)], acc.at[pl.ds(r0, RPT)])
    plsc.subcore_barrier()

    def scat(i, buf):
        pltpu.sync_copy(rows.at[buf], acc.at[didx.at[i]], add=True)

    def step(k, carry):
        i = 3 * k
        pltpu.async_copy(xp.at[sidx.at[i + 2]], rows.at[2], sem2)
        pltpu.make_async_copy(xp.at[sidx.at[i]], rows.at[0], sem0).wait()
        scat(i, 0)
        pltpu.async_copy(xp.at[sidx.at[i + 3]], rows.at[0], sem0)
        pltpu.make_async_copy(xp.at[sidx.at[i + 1]], rows.at[1], sem1).wait()
        scat(i + 1, 1)
        pltpu.async_copy(xp.at[sidx.at[i + 4]], rows.at[1], sem1)
        pltpu.make_async_copy(xp.at[sidx.at[i + 2]], rows.at[2], sem2).wait()
        scat(i + 2, 2)
        return carry

    lax.fori_loop(0, (NITER - 2) // 3, step, 0)
    pltpu.make_async_copy(xp.at[sidx.at[NITER - 2]], rows.at[0], sem0).wait()
    scat(NITER - 2, 0)
    pltpu.make_async_copy(xp.at[sidx.at[NITER - 1]], rows.at[1], sem1).wait()
    scat(NITER - 1, 1)
    plsc.subcore_barrier()

    out_r0 = pl.multiple_of(c * NPAD + s * RPT, 8)
    pltpu.sync_copy(acc.at[pl.ds(r0, RPT)], part_out.at[pl.ds(out_r0, RPT)])


def _tc_a1_body(x_ref, wl, xp_o):
    xp_o[...] = jnp.dot(x_ref[...], wl[...], preferred_element_type=jnp.float32)


def _tc_a2_body(x_ref, wr, wres, bl, br, bres, xr_o, xres_o):
    x = x_ref[...]
    xr_o[...] = jnp.dot(x, wr[...], preferred_element_type=jnp.float32) + bl[...] + br[...]
    xres_o[...] = jnp.dot(x, wres[...], preferred_element_type=jnp.float32) + bres[...]


def _ln_relu(h, g, be):
    mu = jnp.mean(h, axis=1, keepdims=True)
    var = jnp.mean((h - mu) * (h - mu), axis=1, keepdims=True)
    h = (h - mu) * lax.rsqrt(var + 1e-5) * g + be
    return jnp.maximum(h, 0.0)


def _tc_b1_body(pa, pb, da, db, xr, xres, g, be, wl2, x1_o, xp2_o):
    ssum = pa[0] + pb[0]
    deg = jnp.sum(da[0] + db[0], axis=1, keepdims=True)
    rdeg = 1.0 / jnp.maximum(deg, 1.0)
    h = ssum * rdeg + xr[...]
    h = _ln_relu(h, g[...], be[...])
    x1 = xres[...] + h
    x1_o[...] = x1
    xp2_o[...] = jnp.dot(x1, wl2[...], preferred_element_type=jnp.float32)


def _tc_b2_body(x1_ref, wr2, bl2, br2, xr2_o):
    xr2_o[...] = jnp.dot(x1_ref[...], wr2[...], preferred_element_type=jnp.float32) + bl2[...] + br2[...]


def _tc_c_body(pa, pb, da, db, xr2, x1, g, be, wh, bh, out_o):
    ssum = pa[0] + pb[0]
    deg = jnp.sum(da[0] + db[0], axis=1, keepdims=True)
    rdeg = 1.0 / jnp.maximum(deg, 1.0)
    h = ssum * rdeg + xr2[...]
    h = _ln_relu(h, g[...], be[...])
    x2 = x1[...] + h
    out_o[...] = jnp.dot(x2, wh[...], preferred_element_type=jnp.float32) + bh[...]


def _row_spec(width):
    return pl.BlockSpec((BLK, width), lambda i: (i, 0))


def _slab_spec(width, slab):
    return pl.BlockSpec((1, BLK, width), lambda i, _s=slab: (_s, i, 0))


def _full_spec(shape):
    return pl.BlockSpec(shape, lambda i: (0,) * len(shape))


def kernel(x, edge_index, Wl1, bl1, Wr1, br1, g1, be1, Wres1, bres1,
           Wl2, bl2, Wr2, br2, g2, be2, Wh, bh):
    edge4 = edge_index.astype(jnp.int32).reshape(2, NW, NITER, CH)

    z64 = jnp.zeros((NPAD, HID), jnp.float32)
    z16 = jnp.zeros((NPAD, DEGW), jnp.float32)
    ones16 = jnp.zeros((CH, DEGW), jnp.float32).at[:, 0].set(1.0)

    bl1r = bl1.reshape(1, HID)
    br1r = br1.reshape(1, HID)
    bres1r = bres1.reshape(1, HID)
    g1r = g1.reshape(1, HID)
    be1r = be1.reshape(1, HID)
    bl2r = bl2.reshape(1, HID)
    br2r = br2.reshape(1, HID)
    g2r = g2.reshape(1, HID)
    be2r = be2.reshape(1, HID)
    bhr = bh.reshape(1, OUT_DIM)

    xp1 = pl.pallas_call(
        _tc_a1_body,
        grid=(GRID,),
        in_specs=[_row_spec(IN_DIM), _full_spec((IN_DIM, HID))],
        out_specs=_row_spec(HID),
        out_shape=jax.ShapeDtypeStruct((N, HID), jnp.float32),
    )(x, Wl1)

    xr1, xres = pl.pallas_call(
        _tc_a2_body,
        grid=(GRID,),
        in_specs=[
            _row_spec(IN_DIM),
            _full_spec((IN_DIM, HID)),
            _full_spec((IN_DIM, HID)),
            _full_spec((1, HID)),
            _full_spec((1, HID)),
            _full_spec((1, HID)),
        ],
        out_specs=[_row_spec(HID), _row_spec(HID)],
        out_shape=[
            jax.ShapeDtypeStruct((N, HID), jnp.float32),
            jax.ShapeDtypeStruct((N, HID), jnp.float32),
        ],
    )(x, Wr1, Wres1, bl1r, br1r, bres1r)

    part1, degp = _sc_agg_deg(xp1, edge4, z64, z16, ones16)
    part1 = part1.reshape(NC, NPAD, HID)
    degp = degp.reshape(NC, NPAD, DEGW)

    x1, xp2 = pl.pallas_call(
        _tc_b1_body,
        grid=(GRID,),
        in_specs=[
            _slab_spec(HID, 0), _slab_spec(HID, 1),
            _slab_spec(DEGW, 0), _slab_spec(DEGW, 1),
            _row_spec(HID), _row_spec(HID),
            _full_spec((1, HID)), _full_spec((1, HID)),
            _full_spec((HID, HID)),
        ],
        out_specs=[_row_spec(HID), _row_spec(HID)],
        out_shape=[
            jax.ShapeDtypeStruct((N, HID), jnp.float32),
            jax.ShapeDtypeStruct((N, HID), jnp.float32),
        ],
    )(part1, part1, degp, degp, xr1, xres, g1r, be1r, Wl2)

    part2 = _sc_agg(xp2, edge4, z64)
    part2 = part2.reshape(NC, NPAD, HID)

    xr2 = pl.pallas_call(
        _tc_b2_body,
        grid=(GRID,),
        in_specs=[
            _row_spec(HID),
            _full_spec((HID, HID)),
            _full_spec((1, HID)),
            _full_spec((1, HID)),
        ],
        out_specs=_row_spec(HID),
        out_shape=jax.ShapeDtypeStruct((N, HID), jnp.float32),
    )(x1, Wr2, bl2r, br2r)

    out = pl.pallas_call(
        _tc_c_body,
        grid=(GRID,),
        in_specs=[
            _slab_spec(HID, 0), _slab_spec(HID, 1),
            _slab_spec(DEGW, 0), _slab_spec(DEGW, 1),
            _row_spec(HID), _row_spec(HID),
            _full_spec((1, HID)), _full_spec((1, HID)),
            _full_spec((HID, OUT_DIM)), _full_spec((1, OUT_DIM)),
        ],
        out_specs=_row_spec(OUT_DIM),
        out_shape=jax.ShapeDtypeStruct((N, OUT_DIM), jnp.float32),
    )(part2, part2, degp, degp, xr2, x1, g2r, be2r, Wh, bhr)

    return out


# speedup vs baseline: 1.8317x; 1.0708x over previous
import functools

import jax
import jax.numpy as jnp
from jax import lax
from jax.experimental import pallas as pl
from jax.experimental.pallas import tpu as pltpu
from jax.experimental.pallas import tpu_sc as plsc

N = 10000
E = 320000
IN_DIM = 128
HID = 64
OUT_DIM = 128

NC = 2
NS = 16
NW = NC * NS
EPW = E // NW
CH = 80
NITER = EPW // CH
NPAD = 10240
RPT = NPAD // NS
DEGW = 8

BLK = 1000
GRID = N // BLK

_SC_MESH = plsc.VectorSubcoreMesh(core_axis_name="c", subcore_axis_name="s")


@functools.partial(
    pl.kernel,
    out_type=(
        jax.ShapeDtypeStruct((NC * NPAD, HID), jnp.float32),
        jax.ShapeDtypeStruct((NC * NPAD, DEGW), jnp.float32),
    ),
    mesh=_SC_MESH,
    compiler_params=pltpu.CompilerParams(use_tc_tiling_on_sc=False),
    scratch_types=[
        pltpu.VMEM((NITER, CH), jnp.int32),
        pltpu.VMEM((NITER, CH), jnp.int32),
        pltpu.VMEM((4, CH, HID), jnp.float32),
        pltpu.VMEM((CH, DEGW), jnp.float32),
        pltpu.VMEM_SHARED((NPAD, HID), jnp.float32),
        pltpu.VMEM_SHARED((NPAD, DEGW), jnp.float32),
        pltpu.SemaphoreType.DMA,
        pltpu.SemaphoreType.DMA,
        pltpu.SemaphoreType.DMA,
        pltpu.SemaphoreType.DMA,
    ],
)
def _sc_agg_deg(xp, edge4, z64, z16, ones16, part_out, degp_out,
                sidx, didx, rows, ones_v, acc, dacc, sem0, sem1, sem2, sem3):
    c = lax.axis_index("c")
    s = lax.axis_index("s")
    wid = c * NS + s
    r0 = pl.multiple_of(s * RPT, 8)

    pltpu.sync_copy(edge4.at[0].at[wid], sidx)
    pltpu.sync_copy(edge4.at[1].at[wid], didx)
    pltpu.async_copy(xp.at[sidx.at[0]], rows.at[0], sem0)
    pltpu.async_copy(xp.at[sidx.at[1]], rows.at[1], sem1)
    pltpu.async_copy(xp.at[sidx.at[2]], rows.at[2], sem2)
    pltpu.sync_copy(z64.at[pl.ds(r0, RPT)], acc.at[pl.ds(r0, RPT)])
    pltpu.sync_copy(z16.at[pl.ds(r0, RPT)], dacc.at[pl.ds(r0, RPT)])
    pltpu.sync_copy(ones16, ones_v)
    plsc.subcore_barrier()

    def scat(i, buf):
        pltpu.sync_copy(rows.at[buf], acc.at[didx.at[i]], add=True)
        pltpu.sync_copy(ones_v, dacc.at[didx.at[i]], add=True)

    def step(k, carry):
        i = 4 * k
        pltpu.async_copy(xp.at[sidx.at[i + 3]], rows.at[3], sem3)
        pltpu.make_async_copy(xp.at[sidx.at[i]], rows.at[0], sem0).wait()
        scat(i, 0)
        pltpu.async_copy(xp.at[sidx.at[i + 4]], rows.at[0], sem0)
        pltpu.make_async_copy(xp.at[sidx.at[i + 1]], rows.at[1], sem1).wait()
        scat(i + 1, 1)
        pltpu.async_copy(xp.at[sidx.at[i + 5]], rows.at[1], sem1)
        pltpu.make_async_copy(xp.at[sidx.at[i + 2]], rows.at[2], sem2).wait()
        scat(i + 2, 2)
        pltpu.async_copy(xp.at[sidx.at[i + 6]], rows.at[2], sem2)
        pltpu.make_async_copy(xp.at[sidx.at[i + 3]], rows.at[3], sem3).wait()
        scat(i + 3, 3)
        return carry

    NL = (NITER - 5) // 4
    lax.fori_loop(0, NL, step, 0)
    pltpu.make_async_copy(xp.at[sidx.at[NITER - 5]], rows.at[0], sem0).wait()
    scat(NITER - 5, 0)
    pltpu.async_copy(xp.at[sidx.at[NITER - 2]], rows.at[0], sem0)
    pltpu.make_async_copy(xp.at[sidx.at[NITER - 4]], rows.at[1], sem1).wait()
    scat(NITER - 4, 1)
    pltpu.async_copy(xp.at[sidx.at[NITER - 1]], rows.at[1], sem1)
    pltpu.make_async_copy(xp.at[sidx.at[NITER - 3]], rows.at[2], sem2).wait()
    scat(NITER - 3, 2)
    pltpu.make_async_copy(xp.at[sidx.at[NITER - 2]], rows.at[0], sem0).wait()
    scat(NITER - 2, 0)
    pltpu.make_async_copy(xp.at[sidx.at[NITER - 1]], rows.at[1], sem1).wait()
    scat(NITER - 1, 1)
    plsc.subcore_barrier()

    out_r0 = pl.multiple_of(c * NPAD + s * RPT, 8)
    pltpu.sync_copy(acc.at[pl.ds(r0, RPT)], part_out.at[pl.ds(out_r0, RPT)])
    pltpu.sync_copy(dacc.at[pl.ds(r0, RPT)], degp_out.at[pl.ds(out_r0, RPT)])


@functools.partial(
    pl.kernel,
    out_type=jax.ShapeDtypeStruct((NC * NPAD, HID), jnp.float32),
    mesh=_SC_MESH,
    compiler_params=pltpu.CompilerParams(use_tc_tiling_on_sc=False),
    scratch_types=[
        pltpu.VMEM((NITER, CH), jnp.int32),
        pltpu.VMEM((NITER, CH), jnp.int32),
        pltpu.VMEM((4, CH, HID), jnp.float32),
        pltpu.VMEM_SHARED((NPAD, HID), jnp.float32),
        pltpu.SemaphoreType.DMA,
        pltpu.SemaphoreType.DMA,
        pltpu.SemaphoreType.DMA,
        pltpu.SemaphoreType.DMA,
    ],
)
def _sc_agg(xp, edge4, z64, part_out, sidx, didx, rows, acc, sem0, sem1, sem2, sem3):
    c = lax.axis_index("c")
    s = lax.axis_index("s")
    wid = c * NS + s
    r0 = pl.multiple_of(s * RPT, 8)

    pltpu.sync_copy(edge4.at[0].at[wid], sidx)
    pltpu.sync_copy(edge4.at[1].at[wid], didx)
    pltpu.async_copy(xp.at[sidx.at[0]], rows.at[0], sem0)
    pltpu.async_copy(xp.at[sidx.at[1]], rows.at[1], sem1)
    pltpu.async_copy(xp.at[sidx.at[2]], rows.at[2], sem2)
    pltpu.sync_copy(z64.at[pl.ds(r0, RPT)], acc.at[pl.ds(r0, RPT)])
    plsc.subcore_barrier()

    def scat(i, buf):
        pltpu.sync_copy(rows.at[buf], acc.at[didx.at[i]], add=True)

    def step(k, carry):
        i = 4 * k
        pltpu.async_copy(xp.at[sidx.at[i + 3]], rows.at[3], sem3)
        pltpu.make_async_copy(xp.at[sidx.at[i]], rows.at[0], sem0).wait()
        scat(i, 0)
        pltpu.async_copy(xp.at[sidx.at[i + 4]], rows.at[0], sem0)
        pltpu.make_async_copy(xp.at[sidx.at[i + 1]], rows.at[1], sem1).wait()
        scat(i + 1, 1)
        pltpu.async_copy(xp.at[sidx.at[i + 5]], rows.at[1], sem1)
        pltpu.make_async_copy(xp.at[sidx.at[i + 2]], rows.at[2], sem2).wait()
        scat(i + 2, 2)
        pltpu.async_copy(xp.at[sidx.at[i + 6]], rows.at[2], sem2)
        pltpu.make_async_copy(xp.at[sidx.at[i + 3]], rows.at[3], sem3).wait()
        scat(i + 3, 3)
        return carry

    NL = (NITER - 5) // 4
    lax.fori_loop(0, NL, step, 0)
    pltpu.make_async_copy(xp.at[sidx.at[NITER - 5]], rows.at[0], sem0).wait()
    scat(NITER - 5, 0)
    pltpu.async_copy(xp.at[sidx.at[NITER - 2]], rows.at[0], sem0)
    pltpu.make_async_copy(xp.at[sidx.at[NITER - 4]], rows.at[1], sem1).wait()
    scat(NITER - 4, 1)
    pltpu.async_copy(xp.at[sidx.at[NITER - 1]], rows.at[1], sem1)
    pltpu.make_async_copy(xp.at[sidx.at[NITER - 3]], rows.at[2], sem2).wait()
    scat(NITER - 3, 2)
    pltpu.make_async_copy(xp.at[sidx.at[NITER - 2]], rows.at[0], sem0).wait()
    scat(NITER - 2, 0)
    pltpu.make_async_copy(xp.at[sidx.at[NITER - 1]], rows.at[1], sem1).wait()
    scat(NITER - 1, 1)
    plsc.subcore_barrier()

    out_r0 = pl.multiple_of(c * NPAD + s * RPT, 8)
    pltpu.sync_copy(acc.at[pl.ds(r0, RPT)], part_out.at[pl.ds(out_r0, RPT)])


def _tc_a1_body(x_ref, wl, xp_o):
    xp_o[...] = jnp.dot(x_ref[...], wl[...], preferred_element_type=jnp.float32)


def _tc_a2_body(x_ref, wr, wres, bl, br, bres, xr_o, xres_o):
    x = x_ref[...]
    xr_o[...] = jnp.dot(x, wr[...], preferred_element_type=jnp.float32) + bl[...] + br[...]
    xres_o[...] = jnp.dot(x, wres[...], preferred_element_type=jnp.float32) + bres[...]


def _ln_relu(h, g, be):
    mu = jnp.mean(h, axis=1, keepdims=True)
    var = jnp.mean((h - mu) * (h - mu), axis=1, keepdims=True)
    h = (h - mu) * lax.rsqrt(var + 1e-5) * g + be
    return jnp.maximum(h, 0.0)


def _tc_b1_body(pa, pb, da, db, xr, xres, g, be, wl2, x1_o, xp2_o):
    ssum = pa[0] + pb[0]
    deg = jnp.sum(da[0] + db[0], axis=1, keepdims=True)
    rdeg = 1.0 / jnp.maximum(deg, 1.0)
    h = ssum * rdeg + xr[...]
    h = _ln_relu(h, g[...], be[...])
    x1 = xres[...] + h
    x1_o[...] = x1
    xp2_o[...] = jnp.dot(x1, wl2[...], preferred_element_type=jnp.float32)


def _tc_b2_body(x1_ref, wr2, bl2, br2, xr2_o):
    xr2_o[...] = jnp.dot(x1_ref[...], wr2[...], preferred_element_type=jnp.float32) + bl2[...] + br2[...]


def _tc_c_body(pa, pb, da, db, xr2, x1, g, be, wh, bh, out_o):
    ssum = pa[0] + pb[0]
    deg = jnp.sum(da[0] + db[0], axis=1, keepdims=True)
    rdeg = 1.0 / jnp.maximum(deg, 1.0)
    h = ssum * rdeg + xr2[...]
    h = _ln_relu(h, g[...], be[...])
    x2 = x1[...] + h
    out_o[...] = jnp.dot(x2, wh[...], preferred_element_type=jnp.float32) + bh[...]


def _row_spec(width):
    return pl.BlockSpec((BLK, width), lambda i: (i, 0))


def _slab_spec(width, slab):
    return pl.BlockSpec((1, BLK, width), lambda i, _s=slab: (_s, i, 0))


def _full_spec(shape):
    return pl.BlockSpec(shape, lambda i: (0,) * len(shape))


def kernel(x, edge_index, Wl1, bl1, Wr1, br1, g1, be1, Wres1, bres1,
           Wl2, bl2, Wr2, br2, g2, be2, Wh, bh):
    edge4 = edge_index.astype(jnp.int32).reshape(2, NW, NITER, CH)

    z64 = jnp.zeros((NPAD, HID), jnp.float32)
    z16 = jnp.zeros((NPAD, DEGW), jnp.float32)
    ones16 = jnp.zeros((CH, DEGW), jnp.float32).at[:, 0].set(1.0)

    bl1r = bl1.reshape(1, HID)
    br1r = br1.reshape(1, HID)
    bres1r = bres1.reshape(1, HID)
    g1r = g1.reshape(1, HID)
    be1r = be1.reshape(1, HID)
    bl2r = bl2.reshape(1, HID)
    br2r = br2.reshape(1, HID)
    g2r = g2.reshape(1, HID)
    be2r = be2.reshape(1, HID)
    bhr = bh.reshape(1, OUT_DIM)

    xp1 = pl.pallas_call(
        _tc_a1_body,
        grid=(GRID,),
        in_specs=[_row_spec(IN_DIM), _full_spec((IN_DIM, HID))],
        out_specs=_row_spec(HID),
        out_shape=jax.ShapeDtypeStruct((N, HID), jnp.float32),
    )(x, Wl1)

    xr1, xres = pl.pallas_call(
        _tc_a2_body,
        grid=(GRID,),
        in_specs=[
            _row_spec(IN_DIM),
            _full_spec((IN_DIM, HID)),
            _full_spec((IN_DIM, HID)),
            _full_spec((1, HID)),
            _full_spec((1, HID)),
            _full_spec((1, HID)),
        ],
        out_specs=[_row_spec(HID), _row_spec(HID)],
        out_shape=[
            jax.ShapeDtypeStruct((N, HID), jnp.float32),
            jax.ShapeDtypeStruct((N, HID), jnp.float32),
        ],
    )(x, Wr1, Wres1, bl1r, br1r, bres1r)

    part1, degp = _sc_agg_deg(xp1, edge4, z64, z16, ones16)
    part1 = part1.reshape(NC, NPAD, HID)
    degp = degp.reshape(NC, NPAD, DEGW)

    x1, xp2 = pl.pallas_call(
        _tc_b1_body,
        grid=(GRID,),
        in_specs=[
            _slab_spec(HID, 0), _slab_spec(HID, 1),
            _slab_spec(DEGW, 0), _slab_spec(DEGW, 1),
            _row_spec(HID), _row_spec(HID),
            _full_spec((1, HID)), _full_spec((1, HID)),
            _full_spec((HID, HID)),
        ],
        out_specs=[_row_spec(HID), _row_spec(HID)],
        out_shape=[
            jax.ShapeDtypeStruct((N, HID), jnp.float32),
            jax.ShapeDtypeStruct((N, HID), jnp.float32),
        ],
    )(part1, part1, degp, degp, xr1, xres, g1r, be1r, Wl2)

    part2 = _sc_agg(xp2, edge4, z64)
    part2 = part2.reshape(NC, NPAD, HID)

    xr2 = pl.pallas_call(
        _tc_b2_body,
        grid=(GRID,),
        in_specs=[
            _row_spec(HID),
            _full_spec((HID, HID)),
            _full_spec((1, HID)),
            _full_spec((1, HID)),
        ],
        out_specs=_row_spec(HID),
        out_shape=jax.ShapeDtypeStruct((N, HID), jnp.float32),
    )(x1, Wr2, bl2r, br2r)

    out = pl.pallas_call(
        _tc_c_body,
        grid=(GRID,),
        in_specs=[
            _slab_spec(HID, 0), _slab_spec(HID, 1),
            _slab_spec(DEGW, 0), _slab_spec(DEGW, 1),
            _row_spec(HID), _row_spec(HID),
            _full_spec((1, HID)), _full_spec((1, HID)),
            _full_spec((HID, OUT_DIM)), _full_spec((1, OUT_DIM)),
        ],
        out_specs=_row_spec(OUT_DIM),
        out_shape=jax.ShapeDtypeStruct((N, OUT_DIM), jnp.float32),
    )(part2, part2, degp, degp, xr2, x1, g2r, be2r, Wh, bhr)

    return out


# speedup vs baseline: 2.0404x; 1.1139x over previous
import functools

import jax
import jax.numpy as jnp
from jax import lax
from jax.experimental import pallas as pl
from jax.experimental.pallas import tpu as pltpu
from jax.experimental.pallas import tpu_sc as plsc

N = 10000
E = 320000
IN_DIM = 128
HID = 64
OUT_DIM = 128

NC = 2
NS = 16
NW = NC * NS
EPW = E // NW
CH = 80
NITER = EPW // CH
NPAD = 10240
RPT = NPAD // NS
DEGW = 8

BLK = 1000
GRID = N // BLK

_SC_MESH = plsc.VectorSubcoreMesh(core_axis_name="c", subcore_axis_name="s")


@functools.partial(
    pl.kernel,
    out_type=jax.ShapeDtypeStruct((NC * NPAD, 128), jnp.float32),
    mesh=_SC_MESH,
    compiler_params=pltpu.CompilerParams(use_tc_tiling_on_sc=False),
    scratch_types=[
        pltpu.VMEM((NITER, CH), jnp.int32),
        pltpu.VMEM((NITER, CH), jnp.int32),
        pltpu.VMEM((4, CH, HID), jnp.float32),
        pltpu.VMEM((CH, DEGW), jnp.float32),
        pltpu.VMEM_SHARED((NPAD, HID), jnp.float32),
        pltpu.VMEM_SHARED((NPAD, DEGW), jnp.float32),
        pltpu.SemaphoreType.DMA,
        pltpu.SemaphoreType.DMA,
        pltpu.SemaphoreType.DMA,
        pltpu.SemaphoreType.DMA,
    ],
)
def _sc_agg_deg(xp, edge4, z64, z16, ones16, part_out,
                sidx, didx, rows, ones_v, acc, dacc, sem0, sem1, sem2, sem3):
    c = lax.axis_index("c")
    s = lax.axis_index("s")
    wid = c * NS + s
    r0 = pl.multiple_of(s * RPT, 8)

    pltpu.sync_copy(edge4.at[0].at[wid], sidx)
    pltpu.sync_copy(edge4.at[1].at[wid], didx)
    pltpu.async_copy(xp.at[sidx.at[0]], rows.at[0], sem0)
    pltpu.async_copy(xp.at[sidx.at[1]], rows.at[1], sem1)
    pltpu.async_copy(xp.at[sidx.at[2]], rows.at[2], sem2)
    pltpu.sync_copy(z64.at[pl.ds(r0, RPT)], acc.at[pl.ds(r0, RPT)])
    pltpu.sync_copy(z16.at[pl.ds(r0, RPT)], dacc.at[pl.ds(r0, RPT)])
    pltpu.sync_copy(ones16, ones_v)
    plsc.subcore_barrier()

    def scat(i, buf):
        pltpu.sync_copy(rows.at[buf], acc.at[didx.at[i]], add=True)
        pltpu.sync_copy(ones_v, dacc.at[didx.at[i]], add=True)

    def step(k, carry):
        i = 4 * k
        pltpu.async_copy(xp.at[sidx.at[i + 3]], rows.at[3], sem3)
        pltpu.make_async_copy(xp.at[sidx.at[i]], rows.at[0], sem0).wait()
        scat(i, 0)
        pltpu.async_copy(xp.at[sidx.at[i + 4]], rows.at[0], sem0)
        pltpu.make_async_copy(xp.at[sidx.at[i + 1]], rows.at[1], sem1).wait()
        scat(i + 1, 1)
        pltpu.async_copy(xp.at[sidx.at[i + 5]], rows.at[1], sem1)
        pltpu.make_async_copy(xp.at[sidx.at[i + 2]], rows.at[2], sem2).wait()
        scat(i + 2, 2)
        pltpu.async_copy(xp.at[sidx.at[i + 6]], rows.at[2], sem2)
        pltpu.make_async_copy(xp.at[sidx.at[i + 3]], rows.at[3], sem3).wait()
        scat(i + 3, 3)
        return carry

    NL = (NITER - 5) // 4
    lax.fori_loop(0, NL, step, 0)
    pltpu.make_async_copy(xp.at[sidx.at[NITER - 5]], rows.at[0], sem0).wait()
    scat(NITER - 5, 0)
    pltpu.async_copy(xp.at[sidx.at[NITER - 2]], rows.at[0], sem0)
    pltpu.make_async_copy(xp.at[sidx.at[NITER - 4]], rows.at[1], sem1).wait()
    scat(NITER - 4, 1)
    pltpu.async_copy(xp.at[sidx.at[NITER - 1]], rows.at[1], sem1)
    pltpu.make_async_copy(xp.at[sidx.at[NITER - 3]], rows.at[2], sem2).wait()
    scat(NITER - 3, 2)
    pltpu.make_async_copy(xp.at[sidx.at[NITER - 2]], rows.at[0], sem0).wait()
    scat(NITER - 2, 0)
    pltpu.make_async_copy(xp.at[sidx.at[NITER - 1]], rows.at[1], sem1).wait()
    scat(NITER - 1, 1)
    plsc.subcore_barrier()

    out_r0 = pl.multiple_of(c * NPAD + s * RPT, 8)
    pltpu.sync_copy(acc.at[pl.ds(r0, RPT)],
                    part_out.at[pl.ds(out_r0, RPT), pl.ds(0, HID)])
    pltpu.sync_copy(dacc.at[pl.ds(r0, RPT)],
                    part_out.at[pl.ds(out_r0, RPT), pl.ds(HID, DEGW)])


@functools.partial(
    pl.kernel,
    out_type=jax.ShapeDtypeStruct((NC * NPAD, 128), jnp.float32),
    mesh=_SC_MESH,
    compiler_params=pltpu.CompilerParams(use_tc_tiling_on_sc=False),
    scratch_types=[
        pltpu.VMEM((NITER, CH), jnp.int32),
        pltpu.VMEM((NITER, CH), jnp.int32),
        pltpu.VMEM((4, CH, HID), jnp.float32),
        pltpu.VMEM_SHARED((NPAD, HID), jnp.float32),
        pltpu.SemaphoreType.DMA,
        pltpu.SemaphoreType.DMA,
        pltpu.SemaphoreType.DMA,
        pltpu.SemaphoreType.DMA,
    ],
)
def _sc_agg(xp, edge4, z64, part_out, sidx, didx, rows, acc, sem0, sem1, sem2, sem3):
    c = lax.axis_index("c")
    s = lax.axis_index("s")
    wid = c * NS + s
    r0 = pl.multiple_of(s * RPT, 8)

    pltpu.sync_copy(edge4.at[0].at[wid], sidx)
    pltpu.sync_copy(edge4.at[1].at[wid], didx)
    pltpu.async_copy(xp.at[sidx.at[0]], rows.at[0], sem0)
    pltpu.async_copy(xp.at[sidx.at[1]], rows.at[1], sem1)
    pltpu.async_copy(xp.at[sidx.at[2]], rows.at[2], sem2)
    pltpu.sync_copy(z64.at[pl.ds(r0, RPT)], acc.at[pl.ds(r0, RPT)])
    plsc.subcore_barrier()

    def scat(i, buf):
        pltpu.sync_copy(rows.at[buf], acc.at[didx.at[i]], add=True)

    def step(k, carry):
        i = 4 * k
        pltpu.async_copy(xp.at[sidx.at[i + 3]], rows.at[3], sem3)
        pltpu.make_async_copy(xp.at[sidx.at[i]], rows.at[0], sem0).wait()
        scat(i, 0)
        pltpu.async_copy(xp.at[sidx.at[i + 4]], rows.at[0], sem0)
        pltpu.make_async_copy(xp.at[sidx.at[i + 1]], rows.at[1], sem1).wait()
        scat(i + 1, 1)
        pltpu.async_copy(xp.at[sidx.at[i + 5]], rows.at[1], sem1)
        pltpu.make_async_copy(xp.at[sidx.at[i + 2]], rows.at[2], sem2).wait()
        scat(i + 2, 2)
        pltpu.async_copy(xp.at[sidx.at[i + 6]], rows.at[2], sem2)
        pltpu.make_async_copy(xp.at[sidx.at[i + 3]], rows.at[3], sem3).wait()
        scat(i + 3, 3)
        return carry

    NL = (NITER - 5) // 4
    lax.fori_loop(0, NL, step, 0)
    pltpu.make_async_copy(xp.at[sidx.at[NITER - 5]], rows.at[0], sem0).wait()
    scat(NITER - 5, 0)
    pltpu.async_copy(xp.at[sidx.at[NITER - 2]], rows.at[0], sem0)
    pltpu.make_async_copy(xp.at[sidx.at[NITER - 4]], rows.at[1], sem1).wait()
    scat(NITER - 4, 1)
    pltpu.async_copy(xp.at[sidx.at[NITER - 1]], rows.at[1], sem1)
    pltpu.make_async_copy(xp.at[sidx.at[NITER - 3]], rows.at[2], sem2).wait()
    scat(NITER - 3, 2)
    pltpu.make_async_copy(xp.at[sidx.at[NITER - 2]], rows.at[0], sem0).wait()
    scat(NITER - 2, 0)
    pltpu.make_async_copy(xp.at[sidx.at[NITER - 1]], rows.at[1], sem1).wait()
    scat(NITER - 1, 1)
    plsc.subcore_barrier()

    out_r0 = pl.multiple_of(c * NPAD + s * RPT, 8)
    pltpu.sync_copy(acc.at[pl.ds(r0, RPT)],
                    part_out.at[pl.ds(out_r0, RPT), pl.ds(0, HID)])


def _tc_a1_body(x_ref, wl, xp_o):
    xp_o[...] = jnp.dot(x_ref[...], wl[...], preferred_element_type=jnp.float32)


def _tc_a2_body(x_ref, wr, wres, bl, br, bres, xr_o, xres_o):
    x = x_ref[...]
    xr_o[...] = jnp.dot(x, wr[...], preferred_element_type=jnp.float32) + bl[...] + br[...]
    xres_o[...] = jnp.dot(x, wres[...], preferred_element_type=jnp.float32) + bres[...]


def _ln_relu(h, g, be):
    mu = jnp.mean(h, axis=1, keepdims=True)
    var = jnp.mean((h - mu) * (h - mu), axis=1, keepdims=True)
    h = (h - mu) * lax.rsqrt(var + 1e-5) * g + be
    return jnp.maximum(h, 0.0)


def _tc_b1_body(pa, pb, xr, xres, g, be, wl2, x1_o, xp2_o, rdeg_o):
    pa0 = pa[0]
    pb0 = pb[0]
    ssum = pa0[:, :HID] + pb0[:, :HID]
    deg = jnp.sum(pa0[:, HID:HID + DEGW] + pb0[:, HID:HID + DEGW],
                  axis=1, keepdims=True)
    rdeg = 1.0 / jnp.maximum(deg, 1.0)
    rdeg_o[...] = jnp.broadcast_to(rdeg, (BLK, DEGW))
    h = ssum * rdeg + xr[...]
    h = _ln_relu(h, g[...], be[...])
    x1 = xres[...] + h
    x1_o[...] = x1
    xp2_o[...] = jnp.dot(x1, wl2[...], preferred_element_type=jnp.float32)


def _tc_b2_body(x1_ref, wr2, bl2, br2, xr2_o):
    xr2_o[...] = jnp.dot(x1_ref[...], wr2[...], preferred_element_type=jnp.float32) + bl2[...] + br2[...]


def _tc_c_body(pa, pb, rdeg_ref, xr2, x1, g, be, wh, bh, out_o):
    ssum = pa[0][:, :HID] + pb[0][:, :HID]
    rdeg = rdeg_ref[:, 0:1]
    h = ssum * rdeg + xr2[...]
    h = _ln_relu(h, g[...], be[...])
    x2 = x1[...] + h
    out_o[...] = jnp.dot(x2, wh[...], preferred_element_type=jnp.float32) + bh[...]


def _row_spec(width):
    return pl.BlockSpec((BLK, width), lambda i: (i, 0))


def _slab_spec(width, slab):
    return pl.BlockSpec((1, BLK, width), lambda i, _s=slab: (_s, i, 0))


def _full_spec(shape):
    return pl.BlockSpec(shape, lambda i: (0,) * len(shape))


def kernel(x, edge_index, Wl1, bl1, Wr1, br1, g1, be1, Wres1, bres1,
           Wl2, bl2, Wr2, br2, g2, be2, Wh, bh):
    edge4 = edge_index.astype(jnp.int32).reshape(2, NW, NITER, CH)

    z64 = jnp.zeros((NPAD, HID), jnp.float32)
    z16 = jnp.zeros((NPAD, DEGW), jnp.float32)
    ones16 = jnp.zeros((CH, DEGW), jnp.float32).at[:, 0].set(1.0)

    bl1r = bl1.reshape(1, HID)
    br1r = br1.reshape(1, HID)
    bres1r = bres1.reshape(1, HID)
    g1r = g1.reshape(1, HID)
    be1r = be1.reshape(1, HID)
    bl2r = bl2.reshape(1, HID)
    br2r = br2.reshape(1, HID)
    g2r = g2.reshape(1, HID)
    be2r = be2.reshape(1, HID)
    bhr = bh.reshape(1, OUT_DIM)

    xp1 = pl.pallas_call(
        _tc_a1_body,
        grid=(GRID,),
        in_specs=[_row_spec(IN_DIM), _full_spec((IN_DIM, HID))],
        out_specs=_row_spec(HID),
        out_shape=jax.ShapeDtypeStruct((N, HID), jnp.float32),
    )(x, Wl1)

    xr1, xres = pl.pallas_call(
        _tc_a2_body,
        grid=(GRID,),
        in_specs=[
            _row_spec(IN_DIM),
            _full_spec((IN_DIM, HID)),
            _full_spec((IN_DIM, HID)),
            _full_spec((1, HID)),
            _full_spec((1, HID)),
            _full_spec((1, HID)),
        ],
        out_specs=[_row_spec(HID), _row_spec(HID)],
        out_shape=[
            jax.ShapeDtypeStruct((N, HID), jnp.float32),
            jax.ShapeDtypeStruct((N, HID), jnp.float32),
        ],
    )(x, Wr1, Wres1, bl1r, br1r, bres1r)

    part1 = _sc_agg_deg(xp1, edge4, z64, z16, ones16).reshape(NC, NPAD, 128)

    x1, xp2, rdeg = pl.pallas_call(
        _tc_b1_body,
        grid=(GRID,),
        in_specs=[
            _slab_spec(128, 0), _slab_spec(128, 1),
            _row_spec(HID), _row_spec(HID),
            _full_spec((1, HID)), _full_spec((1, HID)),
            _full_spec((HID, HID)),
        ],
        out_specs=[_row_spec(HID), _row_spec(HID), _row_spec(DEGW)],
        out_shape=[
            jax.ShapeDtypeStruct((N, HID), jnp.float32),
            jax.ShapeDtypeStruct((N, HID), jnp.float32),
            jax.ShapeDtypeStruct((N, DEGW), jnp.float32),
        ],
    )(part1, part1, xr1, xres, g1r, be1r, Wl2)

    part2 = _sc_agg(xp2, edge4, z64).reshape(NC, NPAD, 128)

    xr2 = pl.pallas_call(
        _tc_b2_body,
        grid=(GRID,),
        in_specs=[
            _row_spec(HID),
            _full_spec((HID, HID)),
            _full_spec((1, HID)),
            _full_spec((1, HID)),
        ],
        out_specs=_row_spec(HID),
        out_shape=jax.ShapeDtypeStruct((N, HID), jnp.float32),
    )(x1, Wr2, bl2r, br2r)

    out = pl.pallas_call(
        _tc_c_body,
        grid=(GRID,),
        in_specs=[
            _slab_spec(128, 0), _slab_spec(128, 1),
            _row_spec(DEGW),
            _row_spec(HID), _row_spec(HID),
            _full_spec((1, HID)), _full_spec((1, HID)),
            _full_spec((HID, OUT_DIM)), _full_spec((1, OUT_DIM)),
        ],
        out_specs=_row_spec(OUT_DIM),
        out_shape=jax.ShapeDtypeStruct((N, OUT_DIM), jnp.float32),
    )(part2, part2, rdeg, xr2, x1, g2r, be2r, Wh, bhr)

    return out


# speedup vs baseline: 2.0733x; 1.0161x over previous
import functools

import jax
import jax.numpy as jnp
from jax import lax
from jax.experimental import pallas as pl
from jax.experimental.pallas import tpu as pltpu
from jax.experimental.pallas import tpu_sc as plsc

N = 10000
E = 320000
IN_DIM = 128
HID = 64
OUT_DIM = 128

NC = 2
NS = 16
NW = NC * NS
EPW = E // NW
CH = 80
NITER = EPW // CH
NPAD = 10240
RPT = NPAD // NS
DEGW = 8

BLK = 1000
GRID = N // BLK

_SC_MESH = plsc.VectorSubcoreMesh(core_axis_name="c", subcore_axis_name="s")


@functools.partial(
    pl.kernel,
    out_type=jax.ShapeDtypeStruct((NC * NPAD, 128), jnp.float32),
    mesh=_SC_MESH,
    compiler_params=pltpu.CompilerParams(use_tc_tiling_on_sc=False),
    scratch_types=[
        pltpu.VMEM((NITER, CH), jnp.int32),
        pltpu.VMEM((NITER, CH), jnp.int32),
        pltpu.VMEM((6, CH, HID), jnp.float32),
        pltpu.VMEM((CH, DEGW), jnp.float32),
        pltpu.VMEM_SHARED((NPAD, HID), jnp.float32),
        pltpu.VMEM_SHARED((NPAD, DEGW), jnp.float32),
        pltpu.SemaphoreType.DMA,
        pltpu.SemaphoreType.DMA,
        pltpu.SemaphoreType.DMA,
        pltpu.SemaphoreType.DMA,
        pltpu.SemaphoreType.DMA,
        pltpu.SemaphoreType.DMA,
    ],
)
def _sc_agg_deg(xp, edge4, z64, z16, ones16, part_out,
                sidx, didx, rows, ones_v, acc, dacc, *sems):
    c = lax.axis_index("c")
    s = lax.axis_index("s")
    wid = c * NS + s
    r0 = pl.multiple_of(s * RPT, 8)

    pltpu.sync_copy(edge4.at[0].at[wid], sidx)
    pltpu.sync_copy(edge4.at[1].at[wid], didx)
    for _b in range(5):
        pltpu.async_copy(xp.at[sidx.at[_b]], rows.at[_b], sems[_b])
    pltpu.sync_copy(z64.at[pl.ds(r0, RPT)], acc.at[pl.ds(r0, RPT)])
    pltpu.sync_copy(z16.at[pl.ds(r0, RPT)], dacc.at[pl.ds(r0, RPT)])
    pltpu.sync_copy(ones16, ones_v)
    plsc.subcore_barrier()

    def scat(i, buf):
        pltpu.sync_copy(rows.at[buf], acc.at[didx.at[i]], add=True)
        pltpu.sync_copy(ones_v, dacc.at[didx.at[i]], add=True)

    def step(k, carry):
        i = 6 * k
        for _b in range(6):
            pltpu.async_copy(xp.at[sidx.at[i + 5 + _b]],
                             rows.at[(5 + _b) % 6], sems[(5 + _b) % 6])
            pltpu.make_async_copy(xp.at[sidx.at[i + _b]],
                                  rows.at[_b], sems[_b]).wait()
            scat(i + _b, _b)
        return carry

    NL = (NITER - 5) // 6
    lax.fori_loop(0, NL, step, 0)
    for _b in range(5):
        j = NITER - 5 + _b
        pltpu.make_async_copy(xp.at[sidx.at[j]], rows.at[_b], sems[_b]).wait()
        scat(j, _b)
    plsc.subcore_barrier()

    out_r0 = pl.multiple_of(c * NPAD + s * RPT, 8)
    pltpu.sync_copy(acc.at[pl.ds(r0, RPT)],
                    part_out.at[pl.ds(out_r0, RPT), pl.ds(0, HID)])
    pltpu.sync_copy(dacc.at[pl.ds(r0, RPT)],
                    part_out.at[pl.ds(out_r0, RPT), pl.ds(HID, DEGW)])


@functools.partial(
    pl.kernel,
    out_type=jax.ShapeDtypeStruct((NC * NPAD, 128), jnp.float32),
    mesh=_SC_MESH,
    compiler_params=pltpu.CompilerParams(use_tc_tiling_on_sc=False),
    scratch_types=[
        pltpu.VMEM((NITER, CH), jnp.int32),
        pltpu.VMEM((NITER, CH), jnp.int32),
        pltpu.VMEM((6, CH, HID), jnp.float32),
        pltpu.VMEM_SHARED((NPAD, HID), jnp.float32),
        pltpu.SemaphoreType.DMA,
        pltpu.SemaphoreType.DMA,
        pltpu.SemaphoreType.DMA,
        pltpu.SemaphoreType.DMA,
        pltpu.SemaphoreType.DMA,
        pltpu.SemaphoreType.DMA,
    ],
)
def _sc_agg(xp, edge4, z64, part_out, sidx, didx, rows, acc, *sems):
    c = lax.axis_index("c")
    s = lax.axis_index("s")
    wid = c * NS + s
    r0 = pl.multiple_of(s * RPT, 8)

    pltpu.sync_copy(edge4.at[0].at[wid], sidx)
    pltpu.sync_copy(edge4.at[1].at[wid], didx)
    for _b in range(5):
        pltpu.async_copy(xp.at[sidx.at[_b]], rows.at[_b], sems[_b])
    pltpu.sync_copy(z64.at[pl.ds(r0, RPT)], acc.at[pl.ds(r0, RPT)])
    plsc.subcore_barrier()

    def scat(i, buf):
        pltpu.sync_copy(rows.at[buf], acc.at[didx.at[i]], add=True)

    def step(k, carry):
        i = 6 * k
        for _b in range(6):
            pltpu.async_copy(xp.at[sidx.at[i + 5 + _b]],
                             rows.at[(5 + _b) % 6], sems[(5 + _b) % 6])
            pltpu.make_async_copy(xp.at[sidx.at[i + _b]],
                                  rows.at[_b], sems[_b]).wait()
            scat(i + _b, _b)
        return carry

    NL = (NITER - 5) // 6
    lax.fori_loop(0, NL, step, 0)
    for _b in range(5):
        j = NITER - 5 + _b
        pltpu.make_async_copy(xp.at[sidx.at[j]], rows.at[_b], sems[_b]).wait()
        scat(j, _b)
    plsc.subcore_barrier()

    out_r0 = pl.multiple_of(c * NPAD + s * RPT, 8)
    pltpu.sync_copy(acc.at[pl.ds(r0, RPT)],
                    part_out.at[pl.ds(out_r0, RPT), pl.ds(0, HID)])


def _tc_a1_body(x_ref, wl, xp_o):
    xp_o[...] = jnp.dot(x_ref[...], wl[...], preferred_element_type=jnp.float32)


def _tc_a2_body(x_ref, wr, wres, bl, br, bres, xr_o, xres_o):
    x = x_ref[...]
    xr_o[...] = jnp.dot(x, wr[...], preferred_element_type=jnp.float32) + bl[...] + br[...]
    xres_o[...] = jnp.dot(x, wres[...], preferred_element_type=jnp.float32) + bres[...]


def _ln_relu(h, g, be):
    mu = jnp.mean(h, axis=1, keepdims=True)
    var = jnp.mean((h - mu) * (h - mu), axis=1, keepdims=True)
    h = (h - mu) * lax.rsqrt(var + 1e-5) * g + be
    return jnp.maximum(h, 0.0)


def _tc_b1_body(pa, pb, xr, xres, g, be, wl2, x1_o, xp2_o, rdeg_o):
    pa0 = pa[0]
    pb0 = pb[0]
    ssum = pa0[:, :HID] + pb0[:, :HID]
    deg = jnp.sum(pa0[:, HID:HID + DEGW] + pb0[:, HID:HID + DEGW],
                  axis=1, keepdims=True)
    rdeg = 1.0 / jnp.maximum(deg, 1.0)
    rdeg_o[...] = jnp.broadcast_to(rdeg, (BLK, DEGW))
    h = ssum * rdeg + xr[...]
    h = _ln_relu(h, g[...], be[...])
    x1 = xres[...] + h
    x1_o[...] = x1
    xp2_o[...] = jnp.dot(x1, wl2[...], preferred_element_type=jnp.float32)


def _tc_b2_body(x1_ref, wr2, bl2, br2, xr2_o):
    xr2_o[...] = jnp.dot(x1_ref[...], wr2[...], preferred_element_type=jnp.float32) + bl2[...] + br2[...]


def _tc_c_body(pa, pb, rdeg_ref, xr2, x1, g, be, wh, bh, out_o):
    ssum = pa[0][:, :HID] + pb[0][:, :HID]
    rdeg = rdeg_ref[:, 0:1]
    h = ssum * rdeg + xr2[...]
    h = _ln_relu(h, g[...], be[...])
    x2 = x1[...] + h
    out_o[...] = jnp.dot(x2, wh[...], preferred_element_type=jnp.float32) + bh[...]


def _row_spec(width):
    return pl.BlockSpec((BLK, width), lambda i: (i, 0))


def _slab_spec(width, slab, col_blk=0):
    return pl.BlockSpec((1, BLK, width),
                        lambda i, _s=slab, _c=col_blk: (_s, i, _c))


def _full_spec(shape):
    return pl.BlockSpec(shape, lambda i: (0,) * len(shape))


def kernel(x, edge_index, Wl1, bl1, Wr1, br1, g1, be1, Wres1, bres1,
           Wl2, bl2, Wr2, br2, g2, be2, Wh, bh):
    edge4 = edge_index.astype(jnp.int32).reshape(2, NW, NITER, CH)

    z64 = jnp.zeros((NPAD, HID), jnp.float32)
    z16 = jnp.zeros((NPAD, DEGW), jnp.float32)
    ones16 = jnp.zeros((CH, DEGW), jnp.float32).at[:, 0].set(1.0)

    bl1r = bl1.reshape(1, HID)
    br1r = br1.reshape(1, HID)
    bres1r = bres1.reshape(1, HID)
    g1r = g1.reshape(1, HID)
    be1r = be1.reshape(1, HID)
    bl2r = bl2.reshape(1, HID)
    br2r = br2.reshape(1, HID)
    g2r = g2.reshape(1, HID)
    be2r = be2.reshape(1, HID)
    bhr = bh.reshape(1, OUT_DIM)

    xp1 = pl.pallas_call(
        _tc_a1_body,
        grid=(GRID,),
        in_specs=[_row_spec(IN_DIM), _full_spec((IN_DIM, HID))],
        out_specs=_row_spec(HID),
        out_shape=jax.ShapeDtypeStruct((N, HID), jnp.float32),
    )(x, Wl1)

    xr1, xres = pl.pallas_call(
        _tc_a2_body,
        grid=(GRID,),
        in_specs=[
            _row_spec(IN_DIM),
            _full_spec((IN_DIM, HID)),
            _full_spec((IN_DIM, HID)),
            _full_spec((1, HID)),
            _full_spec((1, HID)),
            _full_spec((1, HID)),
        ],
        out_specs=[_row_spec(HID), _row_spec(HID)],
        out_shape=[
            jax.ShapeDtypeStruct((N, HID), jnp.float32),
            jax.ShapeDtypeStruct((N, HID), jnp.float32),
        ],
    )(x, Wr1, Wres1, bl1r, br1r, bres1r)

    part1 = _sc_agg_deg(xp1, edge4, z64, z16, ones16).reshape(NC, NPAD, 128)

    x1, xp2, rdeg = pl.pallas_call(
        _tc_b1_body,
        grid=(GRID,),
        in_specs=[
            _slab_spec(128, 0), _slab_spec(128, 1),
            _row_spec(HID), _row_spec(HID),
            _full_spec((1, HID)), _full_spec((1, HID)),
            _full_spec((HID, HID)),
        ],
        out_specs=[_row_spec(HID), _row_spec(HID), _row_spec(DEGW)],
        out_shape=[
            jax.ShapeDtypeStruct((N, HID), jnp.float32),
            jax.ShapeDtypeStruct((N, HID), jnp.float32),
            jax.ShapeDtypeStruct((N, DEGW), jnp.float32),
        ],
    )(part1, part1, xr1, xres, g1r, be1r, Wl2)

    part2 = _sc_agg(xp2, edge4, z64).reshape(NC, NPAD, 128)

    xr2 = pl.pallas_call(
        _tc_b2_body,
        grid=(GRID,),
        in_specs=[
            _row_spec(HID),
            _full_spec((HID, HID)),
            _full_spec((1, HID)),
            _full_spec((1, HID)),
        ],
        out_specs=_row_spec(HID),
        out_shape=jax.ShapeDtypeStruct((N, HID), jnp.float32),
    )(x1, Wr2, bl2r, br2r)

    out = pl.pallas_call(
        _tc_c_body,
        grid=(GRID,),
        in_specs=[
            _slab_spec(128, 0), _slab_spec(128, 1),
            _row_spec(DEGW),
            _row_spec(HID), _row_spec(HID),
            _full_spec((1, HID)), _full_spec((1, HID)),
            _full_spec((HID, OUT_DIM)), _full_spec((1, OUT_DIM)),
        ],
        out_specs=_row_spec(OUT_DIM),
        out_shape=jax.ShapeDtypeStruct((N, OUT_DIM), jnp.float32),
    )(part2, part2, rdeg, xr2, x1, g2r, be2r, Wh, bhr)

    return out
